# baseline (speedup 1.0000x reference)
import functools

import jax
import jax.numpy as jnp
from jax import lax
from jax.experimental import pallas as pl
from jax.experimental.pallas import tpu as pltpu
from jax.experimental.pallas import tpu_sc as plsc

_NC = 2
_NS = 16
_NW = _NC * _NS
_CHUNK = 128


def _stream_gather(table, idx_hbm, out_hbm, wid, nch, depth, idx_v, bufs,
                   gsems, wsems, didx_hbm=None, didx_v=None):
    pltpu.sync_copy(idx_hbm.at[wid], idx_v)
    if didx_hbm is not None:
        pltpu.sync_copy(didx_hbm.at[wid], didx_v)
    base = wid * nch * _CHUNK

    def _gather(c, k):
        return pltpu.make_async_copy(table.at[idx_v.at[c]], bufs[k], gsems[k])

    def _wb(c, k):
        if didx_hbm is not None:
            dst = out_hbm.at[didx_v.at[c]]
        else:
            dst = out_hbm.at[pl.ds(base + c * _CHUNK, _CHUNK)]
        return pltpu.make_async_copy(bufs[k], dst, wsems[k])

    for k in range(depth):
        _gather(k, k).start()

    def outer(i, carry):
        for k in range(depth):
            c = i * depth + k
            _gather(c, k).wait()
            _wb(c, k).start()

            @pl.when(c + depth < nch)
            def _():
                _wb(c, k).wait()
                _gather(c + depth, k).start()

        return carry

    lax.fori_loop(0, nch // depth, outer, 0, unroll=False)

    for k in range(depth):
        _wb(nch - depth + k, k).wait()


def _sc_gather_body(ent_hbm, rel_hbm, tidx_hbm, sidx_hbm, ridx_hbm,
                    tdest_hbm, sdest_hbm,
                    tails_out, s_out, r_out,
                    iv_t, iv_s, iv_r, dv_t, dv_s, buf0, buf1, buf2, buf3,
                    g0, g1, g2, g3, w0, w1, w2, w3):
    wid = lax.axis_index("s") * _NC + lax.axis_index("c")
    bufs = (buf0, buf1, buf2, buf3)
    gsems = (g0, g1, g2, g3)
    wsems = (w0, w1, w2, w3)
    nch_t = iv_t.shape[0]
    nch_s = iv_s.shape[0]
    nch_r = iv_r.shape[0]

    def depth_for(nch):
        for d in (4, 3, 2, 1):
            if nch % d == 0 and nch >= d:
                return d
        return 1

    _stream_gather(ent_hbm, tidx_hbm, tails_out, wid, nch_t, depth_for(nch_t),
                   iv_t, bufs, gsems, wsems, didx_hbm=tdest_hbm, didx_v=dv_t)
    _stream_gather(ent_hbm, sidx_hbm, s_out, wid, nch_s, depth_for(nch_s),
                   iv_s, bufs, gsems, wsems, didx_hbm=sdest_hbm, didx_v=dv_s)
    _stream_gather(rel_hbm, ridx_hbm, r_out, wid, nch_r, depth_for(nch_r),
                   iv_r, bufs, gsems, wsems, didx_hbm=sdest_hbm, didx_v=dv_s)


def _sc_gather(entity_embeddings, relation_embeddings, tidx, sidx, ridx,
               tdest, sdest, H):
    nch_t = tidx.shape[1]
    nch_s = sidx.shape[1]
    nch_r = ridx.shape[1]
    mesh = plsc.VectorSubcoreMesh(core_axis_name="c", subcore_axis_name="s",
                                  num_cores=_NC, num_subcores=_NS)
    f32 = jnp.float32
    kern = pl.kernel(
        _sc_gather_body,
        out_type=(
            jax.ShapeDtypeStruct((_NW * nch_t * _CHUNK, H), f32),
            jax.ShapeDtypeStruct((_NW * nch_s * _CHUNK, H), f32),
            jax.ShapeDtypeStruct((_NW * nch_r * _CHUNK, H), f32),
        ),
        mesh=mesh,
        scratch_types=[
            pltpu.VMEM((nch_t, _CHUNK), jnp.int32),
            pltpu.VMEM((nch_s, _CHUNK), jnp.int32),
            pltpu.VMEM((nch_r, _CHUNK), jnp.int32),
            pltpu.VMEM((nch_t, _CHUNK), jnp.int32),
            pltpu.VMEM((nch_s, _CHUNK), jnp.int32),
            pltpu.VMEM((_CHUNK, H), f32),
            pltpu.VMEM((_CHUNK, H), f32),
            pltpu.VMEM((_CHUNK, H), f32),
            pltpu.VMEM((_CHUNK, H), f32),
            pltpu.SemaphoreType.DMA,
            pltpu.SemaphoreType.DMA,
            pltpu.SemaphoreType.DMA,
            pltpu.SemaphoreType.DMA,
            pltpu.SemaphoreType.DMA,
            pltpu.SemaphoreType.DMA,
            pltpu.SemaphoreType.DMA,
            pltpu.SemaphoreType.DMA,
        ],
    )
    return kern(entity_embeddings, relation_embeddings, tidx, sidx, ridx,
                tdest, sdest)


def _sc_unperm_body(src_hbm, idx_hbm, out_hbm, iv, buf0, buf1, g0, g1, w0, w1):
    wid = lax.axis_index("s") * _NC + lax.axis_index("c")
    nch = iv.shape[0]
    _stream_gather(src_hbm, idx_hbm, out_hbm, wid, nch, 2 if nch % 2 == 0 else 1,
                   iv, (buf0, buf1), (g0, g1), (w0, w1))


def _sc_unperm(src, idx3):
    nch = idx3.shape[1]
    H = src.shape[1]
    mesh = plsc.VectorSubcoreMesh(core_axis_name="c", subcore_axis_name="s",
                                  num_cores=_NC, num_subcores=_NS)
    kern = pl.kernel(
        _sc_unperm_body,
        out_type=jax.ShapeDtypeStruct((_NW * nch * _CHUNK, H), jnp.float32),
        mesh=mesh,
        scratch_types=[
            pltpu.VMEM((nch, _CHUNK), jnp.int32),
            pltpu.VMEM((_CHUNK, H), jnp.float32),
            pltpu.VMEM((_CHUNK, H), jnp.float32),
            pltpu.SemaphoreType.DMA,
            pltpu.SemaphoreType.DMA,
            pltpu.SemaphoreType.DMA,
            pltpu.SemaphoreType.DMA,
        ],
    )
    return kern(src, idx3)


def _gru_body(ml_ref, tails_ref, s_ref, r_ref, hl_ref, wt_ref, whh_ref,
              bih_ref, bhh_ref, out_ref, gib_ref):
    b = pl.program_id(0)
    t = pl.program_id(1)
    H = out_ref.shape[1]
    f32 = jnp.float32
    bf16 = jnp.bfloat16
    ml = ml_ref[b]

    @pl.when(t == 0)
    def _():
        out_ref[...] = jnp.zeros_like(out_ref)

    @pl.when(jnp.logical_and(t == 0, ml > 0))
    def _():
        s = s_ref[...].astype(bf16)
        r = r_ref[...].astype(bf16)
        gib_ref[...] = (
            jnp.dot(s, wt_ref[0:H, :], preferred_element_type=f32)
            + jnp.dot(r, wt_ref[H:2 * H, :], preferred_element_type=f32)
            + bih_ref[...]
        )

    @pl.when(t < ml)
    def _():
        h = out_ref[...]
        x_t = tails_ref[0].astype(bf16)
        gi = gib_ref[...] + jnp.dot(x_t, wt_ref[2 * H:3 * H, :],
                                    preferred_element_type=f32)
        gh = jnp.dot(h.astype(bf16), whh_ref[...],
                     preferred_element_type=f32) + bhh_ref[...]
        i_r, i_z, i_n = gi[:, :H], gi[:, H:2 * H], gi[:, 2 * H:]
        h_r, h_z, h_n = gh[:, :H], gh[:, H:2 * H], gh[:, 2 * H:]
        rg = 0.5 * jnp.tanh(0.5 * (i_r + h_r)) + 0.5
        z = 0.5 * jnp.tanh(0.5 * (i_z + h_z)) + 0.5
        n = jnp.tanh(i_n + rg * h_n)
        h_new = (1.0 - z) * n + z * h
        m = hl_ref[0] > t
        out_ref[...] = jnp.where(m, h_new, h)


def _gru(tails, s_rows, r_rows, hist_len, maxlens, W_ih, W_hh, b_ih, b_hh, BB):
    T, B, H = tails.shape
    NB = B // BB
    wt = W_ih.T.astype(jnp.bfloat16)
    whh = W_hh.T.astype(jnp.bfloat16)
    bih = b_ih.reshape(1, 3 * H).astype(jnp.float32)
    bhh = b_hh.reshape(1, 3 * H).astype(jnp.float32)
    hl3 = hist_len.astype(jnp.int32).reshape(NB, BB, 1)

    def tails_map(b, t, ml):
        return (jnp.maximum(jnp.minimum(t, ml[b] - 1), 0), b, 0)

    grid = (NB, T)
    return pl.pallas_call(
        _gru_body,
        grid_spec=pltpu.PrefetchScalarGridSpec(
            num_scalar_prefetch=1,
            grid=grid,
            in_specs=[
                pl.BlockSpec((1, BB, H), tails_map),
                pl.BlockSpec((BB, H), lambda b, t, ml: (b, 0)),
                pl.BlockSpec((BB, H), lambda b, t, ml: (b, 0)),
                pl.BlockSpec((1, BB, 1), lambda b, t, ml: (b, 0, 0)),
                pl.BlockSpec((3 * H, 3 * H), lambda b, t, ml: (0, 0)),
                pl.BlockSpec((H, 3 * H), lambda b, t, ml: (0, 0)),
                pl.BlockSpec((1, 3 * H), lambda b, t, ml: (0, 0)),
                pl.BlockSpec((1, 3 * H), lambda b, t, ml: (0, 0)),
            ],
            out_specs=pl.BlockSpec((BB, H), lambda b, t, ml: (b, 0)),
            scratch_shapes=[pltpu.VMEM((BB, 3 * H), jnp.float32)],
        ),
        out_shape=jax.ShapeDtypeStruct((B, H), jnp.float32),
        compiler_params=pltpu.CompilerParams(
            dimension_semantics=("arbitrary", "arbitrary"),
        ),
    )(maxlens, tails, s_rows, r_rows, hl3, wt, whh, bih, bhh)


@jax.jit
def kernel(all_triples, hist_tails, hist_len, entity_embeddings,
           relation_embeddings, W_ih, W_hh, b_ih, b_hh):
    B, T = hist_tails.shape
    H = entity_embeddings.shape[1]

    NSPLIT = 2
    BB = 2048
    BC = B // NSPLIT
    outs = []
    for c in range(NSPLIT):
        sl = slice(c * BC, (c + 1) * BC)
        hl_c = hist_len[sl].astype(jnp.int32)

        i32 = jnp.int32
        kk = jnp.arange(T + 1, dtype=i32)
        eq = hl_c[:, None] == kk[None, :]
        csum = jnp.cumsum(eq.astype(i32), axis=0)
        rank = jnp.sum(jnp.where(eq, csum, 0), axis=1) - 1
        d = jnp.sum(hl_c[:, None] >= kk[None, 1:], axis=0,
                    dtype=i32)
        d_ext = jnp.concatenate([d, jnp.zeros((1,), i32)])
        n_gt = jnp.sum(jnp.where(eq, d_ext[None, :], 0), axis=1)
        pos = n_gt + rank

        hl_p = jnp.sum(jnp.arange(BC, dtype=i32)[:, None] < d[None, :],
                       axis=1, dtype=i32)
        maxlens = hl_p[::BB]

        nch_t = (T * BC) // (_NW * _CHUNK)
        nch_s = BC // (_NW * _CHUNK)
        tidx = hist_tails[sl].T.astype(i32).reshape(_NW, nch_t, _CHUNK)
        sidx = all_triples[sl, 0].astype(i32).reshape(_NW, nch_s, _CHUNK)
        ridx = all_triples[sl, 1].astype(i32).reshape(_NW, nch_s, _CHUNK)
        tdest = (jnp.arange(T, dtype=i32)[:, None] * BC
                 + pos[None, :]).reshape(_NW, nch_t, _CHUNK)
        sdest = pos.reshape(_NW, nch_s, _CHUNK)

        tails_flat, s_rows, r_rows = _sc_gather(
            entity_embeddings, relation_embeddings, tidx, sidx, ridx,
            tdest, sdest, H)
        tails = tails_flat.reshape(T, BC, H)
        out_sorted = _gru(tails, s_rows, r_rows, hl_p, maxlens,
                          W_ih, W_hh, b_ih, b_hh, BB=BB)
        pidx = pos.reshape(_NW, nch_s, _CHUNK)
        outs.append(_sc_unperm(out_sorted, pidx))
    return jnp.concatenate(outs, axis=0)


# speedup vs baseline: 1.0368x; 1.0368x over previous
import functools

import jax
import jax.numpy as jnp
from jax import lax
from jax.experimental import pallas as pl
from jax.experimental.pallas import tpu as pltpu
from jax.experimental.pallas import tpu_sc as plsc

_NC = 2
_NS = 16
_NW = _NC * _NS
_CHUNK = 128


def _stream_gather(table, idx_hbm, out_hbm, wid, nch, depth, idx_v, bufs,
                   gsems, wsems, didx_hbm=None, didx_v=None):
    pltpu.sync_copy(idx_hbm.at[wid], idx_v)
    if didx_hbm is not None:
        pltpu.sync_copy(didx_hbm.at[wid], didx_v)
    base = wid * nch * _CHUNK

    def _gather(c, k):
        return pltpu.make_async_copy(table.at[idx_v.at[c]], bufs[k], gsems[k])

    def _wb(c, k):
        if didx_hbm is not None:
            dst = out_hbm.at[didx_v.at[c]]
        else:
            dst = out_hbm.at[pl.ds(base + c * _CHUNK, _CHUNK)]
        return pltpu.make_async_copy(bufs[k], dst, wsems[k])

    for k in range(depth):
        _gather(k, k).start()

    def outer(i, carry):
        for k in range(depth):
            c = i * depth + k
            _gather(c, k).wait()
            _wb(c, k).start()

            @pl.when(c + depth < nch)
            def _():
                _wb(c, k).wait()
                _gather(c + depth, k).start()

        return carry

    lax.fori_loop(0, nch // depth, outer, 0, unroll=False)

    for k in range(depth):
        _wb(nch - depth + k, k).wait()


def _sc_gather_body(ent_hbm, rel_hbm, tidx_hbm, sidx_hbm, ridx_hbm,
                    tdest_hbm, sdest_hbm,
                    tails_out, s_out, r_out,
                    iv_t, iv_s, iv_r, dv_t, dv_s, buf0, buf1, buf2, buf3, buf4,
                    g0, g1, g2, g3, g4, w0, w1, w2, w3, w4):
    wid = lax.axis_index("s") * _NC + lax.axis_index("c")
    bufs = (buf0, buf1, buf2, buf3, buf4)
    gsems = (g0, g1, g2, g3, g4)
    wsems = (w0, w1, w2, w3, w4)
    nch_t = iv_t.shape[0]
    nch_s = iv_s.shape[0]
    nch_r = iv_r.shape[0]

    def depth_for(nch):
        for d in (5, 4, 3, 2, 1):
            if nch % d == 0 and nch >= d:
                return d
        return 1

    _stream_gather(ent_hbm, tidx_hbm, tails_out, wid, nch_t, depth_for(nch_t),
                   iv_t, bufs, gsems, wsems, didx_hbm=tdest_hbm, didx_v=dv_t)
    _stream_gather(ent_hbm, sidx_hbm, s_out, wid, nch_s, depth_for(nch_s),
                   iv_s, bufs, gsems, wsems, didx_hbm=sdest_hbm, didx_v=dv_s)
    _stream_gather(rel_hbm, ridx_hbm, r_out, wid, nch_r, depth_for(nch_r),
                   iv_r, bufs, gsems, wsems, didx_hbm=sdest_hbm, didx_v=dv_s)


def _sc_gather(entity_embeddings, relation_embeddings, tidx, sidx, ridx,
               tdest, sdest, H):
    nch_t = tidx.shape[1]
    nch_s = sidx.shape[1]
    nch_r = ridx.shape[1]
    mesh = plsc.VectorSubcoreMesh(core_axis_name="c", subcore_axis_name="s",
                                  num_cores=_NC, num_subcores=_NS)
    f32 = jnp.float32
    kern = pl.kernel(
        _sc_gather_body,
        out_type=(
            jax.ShapeDtypeStruct((_NW * nch_t * _CHUNK, H), f32),
            jax.ShapeDtypeStruct((_NW * nch_s * _CHUNK, H), f32),
            jax.ShapeDtypeStruct((_NW * nch_r * _CHUNK, H), f32),
        ),
        mesh=mesh,
        scratch_types=[
            pltpu.VMEM((nch_t, _CHUNK), jnp.int32),
            pltpu.VMEM((nch_s, _CHUNK), jnp.int32),
            pltpu.VMEM((nch_r, _CHUNK), jnp.int32),
            pltpu.VMEM((nch_t, _CHUNK), jnp.int32),
            pltpu.VMEM((nch_s, _CHUNK), jnp.int32),
            pltpu.VMEM((_CHUNK, H), f32),
            pltpu.VMEM((_CHUNK, H), f32),
            pltpu.VMEM((_CHUNK, H), f32),
            pltpu.VMEM((_CHUNK, H), f32),
            pltpu.VMEM((_CHUNK, H), f32),
            pltpu.SemaphoreType.DMA,
            pltpu.SemaphoreType.DMA,
            pltpu.SemaphoreType.DMA,
            pltpu.SemaphoreType.DMA,
            pltpu.SemaphoreType.DMA,
            pltpu.SemaphoreType.DMA,
            pltpu.SemaphoreType.DMA,
            pltpu.SemaphoreType.DMA,
            pltpu.SemaphoreType.DMA,
            pltpu.SemaphoreType.DMA,
        ],
    )
    return kern(entity_embeddings, relation_embeddings, tidx, sidx, ridx,
                tdest, sdest)


def _sc_unperm_body(src_hbm, idx_hbm, out_hbm, iv, buf0, buf1, g0, g1, w0, w1):
    wid = lax.axis_index("s") * _NC + lax.axis_index("c")
    nch = iv.shape[0]
    _stream_gather(src_hbm, idx_hbm, out_hbm, wid, nch, 2 if nch % 2 == 0 else 1,
                   iv, (buf0, buf1), (g0, g1), (w0, w1))


def _sc_unperm(src, idx3):
    nch = idx3.shape[1]
    H = src.shape[1]
    mesh = plsc.VectorSubcoreMesh(core_axis_name="c", subcore_axis_name="s",
                                  num_cores=_NC, num_subcores=_NS)
    kern = pl.kernel(
        _sc_unperm_body,
        out_type=jax.ShapeDtypeStruct((_NW * nch * _CHUNK, H), jnp.float32),
        mesh=mesh,
        scratch_types=[
            pltpu.VMEM((nch, _CHUNK), jnp.int32),
            pltpu.VMEM((_CHUNK, H), jnp.float32),
            pltpu.VMEM((_CHUNK, H), jnp.float32),
            pltpu.SemaphoreType.DMA,
            pltpu.SemaphoreType.DMA,
            pltpu.SemaphoreType.DMA,
            pltpu.SemaphoreType.DMA,
        ],
    )
    return kern(src, idx3)


def _gru_body(ml_ref, tails_ref, s_ref, r_ref, hl_ref, wt_ref, whh_ref,
              bih_ref, bhh_ref, out_ref, gib_ref):
    b = pl.program_id(0)
    t = pl.program_id(1)
    H = out_ref.shape[1]
    f32 = jnp.float32
    bf16 = jnp.bfloat16
    ml = ml_ref[b]

    @pl.when(t == 0)
    def _():
        out_ref[...] = jnp.zeros_like(out_ref)

    @pl.when(jnp.logical_and(t == 0, ml > 0))
    def _():
        s = s_ref[...].astype(bf16)
        r = r_ref[...].astype(bf16)
        gib_ref[...] = (
            jnp.dot(s, wt_ref[0:H, :], preferred_element_type=f32)
            + jnp.dot(r, wt_ref[H:2 * H, :], preferred_element_type=f32)
            + bih_ref[...]
        )

    @pl.when(t < ml)
    def _():
        h = out_ref[...]
        x_t = tails_ref[0].astype(bf16)
        gi = gib_ref[...] + jnp.dot(x_t, wt_ref[2 * H:3 * H, :],
                                    preferred_element_type=f32)
        gh = jnp.dot(h.astype(bf16), whh_ref[...],
                     preferred_element_type=f32) + bhh_ref[...]
        i_r, i_z, i_n = gi[:, :H], gi[:, H:2 * H], gi[:, 2 * H:]
        h_r, h_z, h_n = gh[:, :H], gh[:, H:2 * H], gh[:, 2 * H:]
        rg = 0.5 * jnp.tanh(0.5 * (i_r + h_r)) + 0.5
        z = 0.5 * jnp.tanh(0.5 * (i_z + h_z)) + 0.5
        n = jnp.tanh(i_n + rg * h_n)
        h_new = (1.0 - z) * n + z * h
        m = hl_ref[0] > t
        out_ref[...] = jnp.where(m, h_new, h)


def _gru(tails, s_rows, r_rows, hist_len, maxlens, W_ih, W_hh, b_ih, b_hh, BB):
    T, B, H = tails.shape
    NB = B // BB
    wt = W_ih.T.astype(jnp.bfloat16)
    whh = W_hh.T.astype(jnp.bfloat16)
    bih = b_ih.reshape(1, 3 * H).astype(jnp.float32)
    bhh = b_hh.reshape(1, 3 * H).astype(jnp.float32)
    hl3 = hist_len.astype(jnp.int32).reshape(NB, BB, 1)

    def tails_map(b, t, ml):
        return (jnp.maximum(jnp.minimum(t, ml[b] - 1), 0), b, 0)

    grid = (NB, T)
    return pl.pallas_call(
        _gru_body,
        grid_spec=pltpu.PrefetchScalarGridSpec(
            num_scalar_prefetch=1,
            grid=grid,
            in_specs=[
                pl.BlockSpec((1, BB, H), tails_map),
                pl.BlockSpec((BB, H), lambda b, t, ml: (b, 0)),
                pl.BlockSpec((BB, H), lambda b, t, ml: (b, 0)),
                pl.BlockSpec((1, BB, 1), lambda b, t, ml: (b, 0, 0)),
                pl.BlockSpec((3 * H, 3 * H), lambda b, t, ml: (0, 0)),
                pl.BlockSpec((H, 3 * H), lambda b, t, ml: (0, 0)),
                pl.BlockSpec((1, 3 * H), lambda b, t, ml: (0, 0)),
                pl.BlockSpec((1, 3 * H), lambda b, t, ml: (0, 0)),
            ],
            out_specs=pl.BlockSpec((BB, H), lambda b, t, ml: (b, 0)),
            scratch_shapes=[pltpu.VMEM((BB, 3 * H), jnp.float32)],
        ),
        out_shape=jax.ShapeDtypeStruct((B, H), jnp.float32),
        compiler_params=pltpu.CompilerParams(
            dimension_semantics=("arbitrary", "arbitrary"),
        ),
    )(maxlens, tails, s_rows, r_rows, hl3, wt, whh, bih, bhh)


@jax.jit
def kernel(all_triples, hist_tails, hist_len, entity_embeddings,
           relation_embeddings, W_ih, W_hh, b_ih, b_hh):
    B, T = hist_tails.shape
    H = entity_embeddings.shape[1]

    NSPLIT = 2
    BB = 4096
    BC = B // NSPLIT
    outs = []
    for c in range(NSPLIT):
        sl = slice(c * BC, (c + 1) * BC)
        hl_c = hist_len[sl].astype(jnp.int32)

        i32 = jnp.int32
        kk = jnp.arange(T + 1, dtype=i32)
        eq = hl_c[:, None] == kk[None, :]
        csum = jnp.cumsum(eq.astype(i32), axis=0)
        rank = jnp.sum(jnp.where(eq, csum, 0), axis=1) - 1
        d = jnp.sum(hl_c[:, None] >= kk[None, 1:], axis=0,
                    dtype=i32)
        d_ext = jnp.concatenate([d, jnp.zeros((1,), i32)])
        n_gt = jnp.sum(jnp.where(eq, d_ext[None, :], 0), axis=1)
        pos = n_gt + rank

        hl_p = jnp.sum(jnp.arange(BC, dtype=i32)[:, None] < d[None, :],
                       axis=1, dtype=i32)
        maxlens = hl_p[::BB]

        nch_t = (T * BC) // (_NW * _CHUNK)
        nch_s = BC // (_NW * _CHUNK)
        tidx = hist_tails[sl].T.astype(i32).reshape(_NW, nch_t, _CHUNK)
        sidx = all_triples[sl, 0].astype(i32).reshape(_NW, nch_s, _CHUNK)
        ridx = all_triples[sl, 1].astype(i32).reshape(_NW, nch_s, _CHUNK)
        tdest = (jnp.arange(T, dtype=i32)[:, None] * BC
                 + pos[None, :]).reshape(_NW, nch_t, _CHUNK)
        sdest = pos.reshape(_NW, nch_s, _CHUNK)

        tails_flat, s_rows, r_rows = _sc_gather(
            entity_embeddings, relation_embeddings, tidx, sidx, ridx,
            tdest, sdest, H)
        tails = tails_flat.reshape(T, BC, H)
        out_sorted = _gru(tails, s_rows, r_rows, hl_p, maxlens,
                          W_ih, W_hh, b_ih, b_hh, BB=BB)
        pidx = pos.reshape(_NW, nch_s, _CHUNK)
        outs.append(_sc_unperm(out_sorted, pidx))
    return jnp.concatenate(outs, axis=0)


# speedup vs baseline: 1.0384x; 1.0015x over previous
import functools

import jax
import jax.numpy as jnp
from jax import lax
from jax.experimental import pallas as pl
from jax.experimental.pallas import tpu as pltpu
from jax.experimental.pallas import tpu_sc as plsc

_NC = 2
_NS = 16
_NW = _NC * _NS
_CHUNK = 128


def _stream_gather(table, idx_hbm, out_hbm, wid, nch, depth, idx_v, bufs,
                   gsems, wsems, didx_hbm=None, didx_v=None):
    pltpu.sync_copy(idx_hbm.at[wid], idx_v)
    if didx_hbm is not None:
        pltpu.sync_copy(didx_hbm.at[wid], didx_v)
    base = wid * nch * _CHUNK

    def _gather(c, k):
        return pltpu.make_async_copy(table.at[idx_v.at[c]], bufs[k], gsems[k])

    def _wb(c, k):
        if didx_hbm is not None:
            dst = out_hbm.at[didx_v.at[c]]
        else:
            dst = out_hbm.at[pl.ds(base + c * _CHUNK, _CHUNK)]
        return pltpu.make_async_copy(bufs[k], dst, wsems[k])

    for k in range(depth):
        _gather(k, k).start()

    def outer(i, carry):
        for k in range(depth):
            c = i * depth + k
            _gather(c, k).wait()
            _wb(c, k).start()

            @pl.when(c + depth < nch)
            def _():
                _wb(c, k).wait()
                _gather(c + depth, k).start()

        return carry

    lax.fori_loop(0, nch // depth, outer, 0, unroll=False)

    for k in range(depth):
        _wb(nch - depth + k, k).wait()


def _sc_gather_body(ent_hbm, rel_hbm, tidx_hbm, sidx_hbm, ridx_hbm,
                    tdest_hbm, sdest_hbm,
                    tails_out, s_out, r_out,
                    iv_t, iv_s, iv_r, dv_t, dv_s, buf0, buf1, buf2, buf3, buf4,
                    g0, g1, g2, g3, g4, w0, w1, w2, w3, w4):
    wid = lax.axis_index("s") * _NC + lax.axis_index("c")
    bufs = (buf0, buf1, buf2, buf3, buf4)
    gsems = (g0, g1, g2, g3, g4)
    wsems = (w0, w1, w2, w3, w4)
    nch_t = iv_t.shape[0]
    nch_s = iv_s.shape[0]
    nch_r = iv_r.shape[0]

    def depth_for(nch):
        for d in (5, 4, 3, 2, 1):
            if nch % d == 0 and nch >= d:
                return d
        return 1

    _stream_gather(ent_hbm, tidx_hbm, tails_out, wid, nch_t, depth_for(nch_t),
                   iv_t, bufs, gsems, wsems, didx_hbm=tdest_hbm, didx_v=dv_t)
    _stream_gather(ent_hbm, sidx_hbm, s_out, wid, nch_s, depth_for(nch_s),
                   iv_s, bufs, gsems, wsems, didx_hbm=sdest_hbm, didx_v=dv_s)
    _stream_gather(rel_hbm, ridx_hbm, r_out, wid, nch_r, depth_for(nch_r),
                   iv_r, bufs, gsems, wsems, didx_hbm=sdest_hbm, didx_v=dv_s)


def _sc_gather(entity_embeddings, relation_embeddings, tidx, sidx, ridx,
               tdest, sdest, H):
    nch_t = tidx.shape[1]
    nch_s = sidx.shape[1]
    nch_r = ridx.shape[1]
    mesh = plsc.VectorSubcoreMesh(core_axis_name="c", subcore_axis_name="s",
                                  num_cores=_NC, num_subcores=_NS)
    f32 = jnp.float32
    kern = pl.kernel(
        _sc_gather_body,
        out_type=(
            jax.ShapeDtypeStruct((_NW * nch_t * _CHUNK, H), f32),
            jax.ShapeDtypeStruct((_NW * nch_s * _CHUNK, H), f32),
            jax.ShapeDtypeStruct((_NW * nch_r * _CHUNK, H), f32),
        ),
        mesh=mesh,
        scratch_types=[
            pltpu.VMEM((nch_t, _CHUNK), jnp.int32),
            pltpu.VMEM((nch_s, _CHUNK), jnp.int32),
            pltpu.VMEM((nch_r, _CHUNK), jnp.int32),
            pltpu.VMEM((nch_t, _CHUNK), jnp.int32),
            pltpu.VMEM((nch_s, _CHUNK), jnp.int32),
            pltpu.VMEM((_CHUNK, H), f32),
            pltpu.VMEM((_CHUNK, H), f32),
            pltpu.VMEM((_CHUNK, H), f32),
            pltpu.VMEM((_CHUNK, H), f32),
            pltpu.VMEM((_CHUNK, H), f32),
            pltpu.SemaphoreType.DMA,
            pltpu.SemaphoreType.DMA,
            pltpu.SemaphoreType.DMA,
            pltpu.SemaphoreType.DMA,
            pltpu.SemaphoreType.DMA,
            pltpu.SemaphoreType.DMA,
            pltpu.SemaphoreType.DMA,
            pltpu.SemaphoreType.DMA,
            pltpu.SemaphoreType.DMA,
            pltpu.SemaphoreType.DMA,
        ],
    )
    return kern(entity_embeddings, relation_embeddings, tidx, sidx, ridx,
                tdest, sdest)


def _sc_unperm_body(src_hbm, idx_hbm, out_hbm, iv, buf0, buf1, g0, g1, w0, w1):
    wid = lax.axis_index("s") * _NC + lax.axis_index("c")
    nch = iv.shape[0]
    _stream_gather(src_hbm, idx_hbm, out_hbm, wid, nch, 2 if nch % 2 == 0 else 1,
                   iv, (buf0, buf1), (g0, g1), (w0, w1))


def _sc_unperm(src, idx3):
    nch = idx3.shape[1]
    H = src.shape[1]
    mesh = plsc.VectorSubcoreMesh(core_axis_name="c", subcore_axis_name="s",
                                  num_cores=_NC, num_subcores=_NS)
    kern = pl.kernel(
        _sc_unperm_body,
        out_type=jax.ShapeDtypeStruct((_NW * nch * _CHUNK, H), jnp.float32),
        mesh=mesh,
        scratch_types=[
            pltpu.VMEM((nch, _CHUNK), jnp.int32),
            pltpu.VMEM((_CHUNK, H), jnp.float32),
            pltpu.VMEM((_CHUNK, H), jnp.float32),
            pltpu.SemaphoreType.DMA,
            pltpu.SemaphoreType.DMA,
            pltpu.SemaphoreType.DMA,
            pltpu.SemaphoreType.DMA,
        ],
    )
    return kern(src, idx3)


def _gru_body(ml_ref, tails_ref, s_ref, r_ref, hl_ref, wt_ref, whh_ref,
              bih_ref, bhh_ref, out_ref, gib_ref):
    b = pl.program_id(0)
    t = pl.program_id(1)
    H = out_ref.shape[1]
    f32 = jnp.float32
    bf16 = jnp.bfloat16
    ml = ml_ref[b]

    @pl.when(t == 0)
    def _():
        out_ref[...] = jnp.zeros_like(out_ref)

    @pl.when(jnp.logical_and(t == 0, ml > 0))
    def _():
        s = s_ref[...].astype(bf16)
        r = r_ref[...].astype(bf16)
        gib_ref[...] = (
            jnp.dot(s, wt_ref[0:H, :], preferred_element_type=f32)
            + jnp.dot(r, wt_ref[H:2 * H, :], preferred_element_type=f32)
            + bih_ref[...]
        )

    @pl.when(t < ml)
    def _():
        h = out_ref[...]
        x_t = tails_ref[0].astype(bf16)
        gi = gib_ref[...] + jnp.dot(x_t, wt_ref[2 * H:3 * H, :],
                                    preferred_element_type=f32)
        gh = jnp.dot(h.astype(bf16), whh_ref[...],
                     preferred_element_type=f32) + bhh_ref[...]
        i_r, i_z, i_n = gi[:, :H], gi[:, H:2 * H], gi[:, 2 * H:]
        h_r, h_z, h_n = gh[:, :H], gh[:, H:2 * H], gh[:, 2 * H:]
        rg = 0.5 * jnp.tanh(0.5 * (i_r + h_r)) + 0.5
        z = 0.5 * jnp.tanh(0.5 * (i_z + h_z)) + 0.5
        n = jnp.tanh(i_n + rg * h_n)
        h_new = (1.0 - z) * n + z * h
        m = hl_ref[0] > t
        out_ref[...] = jnp.where(m, h_new, h)


def _gru(tails, s_rows, r_rows, hist_len, maxlens, W_ih, W_hh, b_ih, b_hh, BB):
    T, B, H = tails.shape
    NB = B // BB
    wt = W_ih.T.astype(jnp.bfloat16)
    whh = W_hh.T.astype(jnp.bfloat16)
    bih = b_ih.reshape(1, 3 * H).astype(jnp.float32)
    bhh = b_hh.reshape(1, 3 * H).astype(jnp.float32)
    hl3 = hist_len.astype(jnp.int32).reshape(NB, BB, 1)

    def tails_map(b, t, ml):
        return (jnp.maximum(jnp.minimum(t, ml[b] - 1), 0), b, 0)

    grid = (NB, T)
    return pl.pallas_call(
        _gru_body,
        grid_spec=pltpu.PrefetchScalarGridSpec(
            num_scalar_prefetch=1,
            grid=grid,
            in_specs=[
                pl.BlockSpec((1, BB, H), tails_map),
                pl.BlockSpec((BB, H), lambda b, t, ml: (b, 0)),
                pl.BlockSpec((BB, H), lambda b, t, ml: (b, 0)),
                pl.BlockSpec((1, BB, 1), lambda b, t, ml: (b, 0, 0)),
                pl.BlockSpec((3 * H, 3 * H), lambda b, t, ml: (0, 0)),
                pl.BlockSpec((H, 3 * H), lambda b, t, ml: (0, 0)),
                pl.BlockSpec((1, 3 * H), lambda b, t, ml: (0, 0)),
                pl.BlockSpec((1, 3 * H), lambda b, t, ml: (0, 0)),
            ],
            out_specs=pl.BlockSpec((BB, H), lambda b, t, ml: (b, 0)),
            scratch_shapes=[pltpu.VMEM((BB, 3 * H), jnp.float32)],
        ),
        out_shape=jax.ShapeDtypeStruct((B, H), jnp.float32),
        compiler_params=pltpu.CompilerParams(
            dimension_semantics=("arbitrary", "arbitrary"),
        ),
    )(maxlens, tails, s_rows, r_rows, hl3, wt, whh, bih, bhh)


@jax.jit
def kernel(all_triples, hist_tails, hist_len, entity_embeddings,
           relation_embeddings, W_ih, W_hh, b_ih, b_hh):
    B, T = hist_tails.shape
    H = entity_embeddings.shape[1]

    NSPLIT = 2
    BB = 4096
    BC = B // NSPLIT
    outs = []
    gathered = []
    meta = []
    for c in range(NSPLIT):
        sl = slice(c * BC, (c + 1) * BC)
        hl_c = hist_len[sl].astype(jnp.int32)

        i32 = jnp.int32
        kk = jnp.arange(T + 1, dtype=i32)
        eq = hl_c[:, None] == kk[None, :]
        csum = jnp.cumsum(eq.astype(i32), axis=0)
        rank = jnp.sum(jnp.where(eq, csum, 0), axis=1) - 1
        d = jnp.sum(hl_c[:, None] >= kk[None, 1:], axis=0,
                    dtype=i32)
        d_ext = jnp.concatenate([d, jnp.zeros((1,), i32)])
        n_gt = jnp.sum(jnp.where(eq, d_ext[None, :], 0), axis=1)
        pos = n_gt + rank

        hl_p = jnp.sum(jnp.arange(BC, dtype=i32)[:, None] < d[None, :],
                       axis=1, dtype=i32)
        maxlens = hl_p[::BB]

        nch_t = (T * BC) // (_NW * _CHUNK)
        nch_s = BC // (_NW * _CHUNK)
        tidx = hist_tails[sl].T.astype(i32).reshape(_NW, nch_t, _CHUNK)
        sidx = all_triples[sl, 0].astype(i32).reshape(_NW, nch_s, _CHUNK)
        ridx = all_triples[sl, 1].astype(i32).reshape(_NW, nch_s, _CHUNK)
        tdest = (jnp.arange(T, dtype=i32)[:, None] * BC
                 + pos[None, :]).reshape(_NW, nch_t, _CHUNK)
        sdest = pos.reshape(_NW, nch_s, _CHUNK)

        gathered.append(_sc_gather(
            entity_embeddings, relation_embeddings, tidx, sidx, ridx,
            tdest, sdest, H))
        meta.append((hl_p, maxlens, pos.reshape(_NW, nch_s, _CHUNK)))

    for c in range(NSPLIT):
        tails_flat, s_rows, r_rows = gathered[c]
        hl_p, maxlens, pidx = meta[c]
        tails = tails_flat.reshape(T, BC, H)
        out_sorted = _gru(tails, s_rows, r_rows, hl_p, maxlens,
                          W_ih, W_hh, b_ih, b_hh, BB=BB)
        outs.append(_sc_unperm(out_sorted, pidx))
    return jnp.concatenate(outs, axis=0)
